# baseline (speedup 1.0000x reference)
import numpy as np

import jax
import jax.numpy as jnp
from jax import lax
from jax.experimental import pallas as pl
from jax.experimental.pallas import tpu as pltpu
from jax.experimental.pallas import tpu_sc as plsc

N = 10000
D = 128
E = 320000
ALPHA = 0.5
SLOPE = 0.2
EPS = 1e-5

NC = 2
NS = 16
CH = 128
SCCH = 20
NBUF = 5
NP = 10240
EP = NC * NS * 4 * SCCH * CH
NSCP = EP // (SCCH * CH)
SC_PER_CORE = NSCP // NC
ROWS_PER_TILE = NP // NS
ZROWS = 128
BF = jnp.bfloat16

_QP = np.arange(D).reshape(D // 32, 32)
_QPERM = np.concatenate([_QP[:, 0::2], _QP[:, 1::2]], axis=1).reshape(D)
_INVQ = np.argsort(_QPERM)


def _rsqrt16(x):
    i = lax.bitcast_convert_type(x, jnp.int32)
    i = 0x5F3759DF - lax.shift_right_logical(i, 1)
    y = lax.bitcast_convert_type(i, jnp.float32)
    for _ in range(3):
        y = y * (1.5 - 0.5 * x * y * y)
    return y


def _ln_rows_chunk(buf0, buf1, fstage, wbuf, bbuf, mixstage):
    def rbody(r, _):
        hs = []
        for f in range(D // 32):
            hv = buf0[r, pl.ds(f * 32, 32)] + buf1[r, pl.ds(f * 32, 32)]
            a, b2 = plsc.unpack(hv, format=plsc.PackFormat.INTERLEAVED)
            hs.append(a)
            hs.append(b2)
        hs = [jnp.where(h >= 0, h, SLOPE * h) for h in hs]
        tot = hs[0]
        for h in hs[1:]:
            tot = tot + h
        mu = jnp.sum(tot) * (1.0 / D)
        dvs = [h - mu for h in hs]
        tot2 = dvs[0] * dvs[0]
        for dv in dvs[1:]:
            tot2 = tot2 + dv * dv
        var = jnp.sum(tot2) * (1.0 / D)
        rv = _rsqrt16(jnp.full((16,), var + EPS))
        ys = []
        for i, dv in enumerate(dvs):
            w_ = wbuf[0, pl.ds(i * 16, 16)]
            b_ = bbuf[0, pl.ds(i * 16, 16)]
            x_ = fstage[r, pl.ds(i * 16, 16)]
            y = dv * rv * w_ + b_ + x_
            if mixstage is not None:
                y = (1.0 - ALPHA) * y + ALPHA * mixstage[r, pl.ds(i * 16, 16)]
            ys.append(y)
        for i, y in enumerate(ys):
            fstage[r, pl.ds(i * 16, 16)] = y
        for f in range(D // 32):
            buf0[r, pl.ds(f * 32, 32)] = plsc.pack(
                ys[2 * f], ys[2 * f + 1],
                format=plsc.PackFormat.INTERLEAVED)
        return 0
    lax.fori_loop(0, CH, rbody, 0)


def _phase_common(table, gq, sq, vq, out,
                  gidx_v, sidx_v, val_v, bufs, zbuf, acc, gsem, ssem,
                  c, s, myrow):
    buf0, buf1, buf2 = bufs[0], bufs[1], bufs[2]
    coff = c * NP

    def _zfill(r, _):
        for f in range(D // 32):
            zbuf[r, pl.ds(f * 32, 32)] = jnp.zeros((32,), BF)
        return 0
    lax.fori_loop(0, ZROWS, _zfill, 0)
    for j in range(ROWS_PER_TILE // ZROWS):
        pltpu.sync_copy(zbuf, acc.at[pl.ds(myrow + j * ZROWS, ZROWS)])
    plsc.subcore_barrier()

    def scale(buf, k):
        def sbody(i, _):
            base = pl.multiple_of(i * 16, 16)
            val16 = val_v[k, pl.ds(base, 16)]
            for ee in range(16):
                v16 = jnp.full((16,), val16[ee])
                v = plsc.pack(v16, v16, format=plsc.PackFormat.INTERLEAVED)
                e = base + ee
                for f in range(D // 32):
                    buf[e, pl.ds(f * 32, 32)] = buf[e, pl.ds(f * 32, 32)] * v
            return 0
        lax.fori_loop(0, CH // 16, sbody, 0)

    def outer(j, _):
        sc = c * SC_PER_CORE + j * NS + s
        pltpu.sync_copy(gq.at[sc], gidx_v)
        pltpu.sync_copy(sq.at[sc], sidx_v)
        pltpu.sync_copy(vq.at[sc], val_v)

        def adj(k, _):
            for f in range(CH // 16):
                gidx_v[k, pl.ds(f * 16, 16)] = (
                    gidx_v[k, pl.ds(f * 16, 16)] + coff)
            return 0
        lax.fori_loop(0, SCCH, adj, 0)

        pltpu.make_async_copy(table.at[gidx_v.at[0]], buf0, gsem).start()
        pltpu.make_async_copy(table.at[gidx_v.at[1]], buf1, gsem).start()
        pltpu.make_async_copy(table.at[gidx_v.at[2]], buf2, gsem).start()

        def inner(k4, _):
            for b in range(NBUF):
                k = k4 * NBUF + b
                @pl.when(k >= 2)
                def _():
                    pltpu.make_async_copy(
                        bufs[b], acc.at[sidx_v.at[0]], ssem).wait()

                @pl.when(k < SCCH - 3)
                def _():
                    pltpu.make_async_copy(
                        table.at[gidx_v.at[k + 3]], bufs[(b + 3) % NBUF],
                        gsem).start()
                pltpu.make_async_copy(table.at[gidx_v.at[k]], bufs[b],
                                      gsem).wait()
                scale(bufs[b], k)
                pltpu.make_async_copy(
                    bufs[b], acc.at[sidx_v.at[k]], ssem).start(add=True)
            return 0
        lax.fori_loop(0, SCCH // NBUF, inner, 0)
        pltpu.make_async_copy(buf0, acc.at[sidx_v.at[0]], ssem).wait()
        pltpu.make_async_copy(buf1, acc.at[sidx_v.at[0]], ssem).wait()
        return 0
    lax.fori_loop(0, 4, outer, 0)

    plsc.subcore_barrier()
    pltpu.sync_copy(acc.at[pl.ds(myrow, ROWS_PER_TILE)],
                    out.at[c, pl.ds(myrow, ROWS_PER_TILE)])


def _sc_phase_a_body(table, gq, sq, vq, out,
                     gidx_v, sidx_v, val_v, buf0, buf1, buf2, buf3, buf4,
                     zbuf, fstage, mstage, wbuf, bbuf, acc, gsem, ssem):
    c = lax.axis_index("c")
    s = lax.axis_index("s")
    myrow = s * ROWS_PER_TILE
    _phase_common(table, gq, sq, vq, out,
                  gidx_v, sidx_v, val_v, (buf0, buf1, buf2, buf3, buf4),
                  zbuf, acc, gsem, ssem, c, s, myrow)


def _sc_phase_b_body(p, gq, sq, vq, out, tout,
                     gidx_v, sidx_v, val_v, buf0, buf1, buf2, buf3, buf4,
                     zbuf, fstage, mstage, wbuf, bbuf, acc, gsem, ssem):
    c = lax.axis_index("c")
    s = lax.axis_index("s")
    myrow = s * ROWS_PER_TILE
    for j in range(ROWS_PER_TILE // CH):
        r0 = myrow + j * CH
        pltpu.sync_copy(p.at[0, pl.ds(r0, CH)], buf0)
        pltpu.sync_copy(p.at[1, pl.ds(r0, CH)], buf1)

        def cadd(r, _):
            for f in range(D // 32):
                buf0[r, pl.ds(f * 32, 32)] = (
                    buf0[r, pl.ds(f * 32, 32)] + buf1[r, pl.ds(f * 32, 32)])
            return 0
        lax.fori_loop(0, CH, cadd, 0)
        pltpu.sync_copy(buf0, tout.at[pl.ds(c * NP + r0, CH)])
    _phase_common(tout, gq, sq, vq, out,
                  gidx_v, sidx_v, val_v, (buf0, buf1, buf2, buf3, buf4),
                  zbuf, acc, gsem, ssem, c, s, myrow)


def _sc_phase_bln_body(p, xq, wq, bq, gq, sq, vq, out, tout, xeq,
                       gidx_v, sidx_v, val_v, buf0, buf1, buf2, buf3, buf4,
                       zbuf, fstage, mstage, wbuf, bbuf, acc, gsem, ssem):
    c = lax.axis_index("c")
    s = lax.axis_index("s")
    myrow = s * ROWS_PER_TILE
    pltpu.sync_copy(wq, wbuf)
    pltpu.sync_copy(bq, bbuf)
    for j in range(ROWS_PER_TILE // CH):
        r0 = myrow + j * CH
        pltpu.sync_copy(p.at[0, pl.ds(r0, CH)], buf0)
        pltpu.sync_copy(p.at[1, pl.ds(r0, CH)], buf1)
        pltpu.sync_copy(xq.at[pl.ds(r0, CH)], fstage)
        _ln_rows_chunk(buf0, buf1, fstage, wbuf, bbuf, None)
        pltpu.sync_copy(buf0, tout.at[pl.ds(c * NP + r0, CH)])

        @pl.when(c == 0)
        def _():
            pltpu.sync_copy(fstage, xeq.at[pl.ds(r0, CH)])
    _phase_common(tout, gq, sq, vq, out,
                  gidx_v, sidx_v, val_v, (buf0, buf1, buf2, buf3, buf4),
                  zbuf, acc, gsem, ssem, c, s, myrow)


def _sc_final_body(p, xeq, x0q, wq, bq, xoutq,
                   gidx_v, sidx_v, val_v, buf0, buf1, buf2, buf3, buf4,
                   zbuf, fstage, mstage, wbuf, bbuf, acc, gsem, ssem):
    c = lax.axis_index("c")
    s = lax.axis_index("s")
    myrow = s * ROWS_PER_TILE
    pltpu.sync_copy(wq, wbuf)
    pltpu.sync_copy(bq, bbuf)
    for j in range(ROWS_PER_TILE // CH):
        r0 = myrow + j * CH
        pltpu.sync_copy(p.at[0, pl.ds(r0, CH)], buf0)
        pltpu.sync_copy(p.at[1, pl.ds(r0, CH)], buf1)
        pltpu.sync_copy(xeq.at[pl.ds(r0, CH)], fstage)
        pltpu.sync_copy(x0q.at[pl.ds(r0, CH)], mstage)
        _ln_rows_chunk(buf0, buf1, fstage, wbuf, bbuf, mstage)

        @pl.when(c == 0)
        def _():
            pltpu.sync_copy(fstage, xoutq.at[pl.ds(r0, CH)])


_SCRATCH = [
    pltpu.VMEM((SCCH, CH), jnp.int32),
    pltpu.VMEM((SCCH, CH), jnp.int32),
    pltpu.VMEM((SCCH, CH), jnp.float32),
    pltpu.VMEM((CH, D), BF),
    pltpu.VMEM((CH, D), BF),
    pltpu.VMEM((CH, D), BF),
    pltpu.VMEM((CH, D), BF),
    pltpu.VMEM((CH, D), BF),
    pltpu.VMEM((ZROWS, D), BF),
    pltpu.VMEM((CH, D), jnp.float32),
    pltpu.VMEM((CH, D), jnp.float32),
    pltpu.VMEM((1, D), jnp.float32),
    pltpu.VMEM((1, D), jnp.float32),
    pltpu.VMEM_SHARED((NP, D), BF),
    pltpu.SemaphoreType.DMA,
    pltpu.SemaphoreType.DMA,
]


def _mesh():
    return plsc.VectorSubcoreMesh(core_axis_name="c", subcore_axis_name="s",
                                  num_cores=NC, num_subcores=NS)


def _params():
    return pltpu.CompilerParams(use_tc_tiling_on_sc=False,
                                needs_layout_passes=False)


def _sc_phase_a(table, gq, sq, vq):
    f = pl.kernel(
        _sc_phase_a_body,
        out_type=jax.ShapeDtypeStruct((NC, NP, D), BF),
        mesh=_mesh(), scratch_types=list(_SCRATCH),
        compiler_params=_params(),
    )
    return f(table, gq, sq, vq)


def _sc_phase_b(p, gq, sq, vq):
    f = pl.kernel(
        _sc_phase_b_body,
        out_type=(jax.ShapeDtypeStruct((NC, NP, D), BF),
                  jax.ShapeDtypeStruct((NC * NP, D), BF)),
        mesh=_mesh(), scratch_types=list(_SCRATCH),
        compiler_params=_params(),
    )
    out, _ = f(p, gq, sq, vq)
    return out


def _sc_phase_bln(p, xq, wq, bq, gq, sq, vq):
    f = pl.kernel(
        _sc_phase_bln_body,
        out_type=(jax.ShapeDtypeStruct((NC, NP, D), BF),
                  jax.ShapeDtypeStruct((NC * NP, D), BF),
                  jax.ShapeDtypeStruct((NP, D), jnp.float32)),
        mesh=_mesh(), scratch_types=list(_SCRATCH),
        compiler_params=_params(),
    )
    out, _, xeq = f(p, xq, wq, bq, gq, sq, vq)
    return out, xeq


def _sc_final(p, xeq, x0q, wq, bq):
    f = pl.kernel(
        _sc_final_body,
        out_type=jax.ShapeDtypeStruct((NP, D), jnp.float32),
        mesh=_mesh(), scratch_types=list(_SCRATCH),
        compiler_params=_params(),
    )
    return f(p, xeq, x0q, wq, bq)


_BM = 1280
_NB = NP // _BM


def _tc_dup_cast_body(x_ref, o_ref):
    o_ref[...] = x_ref[...].astype(BF)


def _tc_dup_cast(x):
    return pl.pallas_call(
        _tc_dup_cast_body,
        grid=(NC, _NB),
        in_specs=[pl.BlockSpec((_BM, D), lambda c, i: (i, 0))],
        out_specs=pl.BlockSpec((_BM, D), lambda c, i: (c * _NB + i, 0)),
        out_shape=jax.ShapeDtypeStruct((NC * NP, D), BF),
    )(x)


def _pad_rows(x):
    return jnp.concatenate([x, jnp.zeros((NP - N, D), jnp.float32)], axis=0)


def _pad_edges_idx(x):
    spread = jnp.arange(EP - E, dtype=jnp.int32) % N
    return jnp.concatenate([x, spread]).reshape(NSCP, SCCH, CH)


def kernel(X, adj_indices, adj_values, X0, ln0_w, ln0_b, ln1_w, ln1_b):
    rows3 = _pad_edges_idx(adj_indices[0])
    cols3 = _pad_edges_idx(adj_indices[1])
    vals3 = jnp.concatenate(
        [adj_values, jnp.zeros((EP - E,), jnp.float32)]).reshape(
            NSCP, SCCH, CH)
    qp = jnp.asarray(_QPERM)
    Xp = _pad_rows(X)
    Xq = jnp.take(Xp, qp, axis=1)
    X0q = jnp.take(_pad_rows(X0), qp, axis=1)
    wq0 = jnp.take(ln0_w, qp).reshape(1, D)
    bq0 = jnp.take(ln0_b, qp).reshape(1, D)
    wq1 = jnp.take(ln1_w, qp).reshape(1, D)
    bq1 = jnp.take(ln1_b, qp).reshape(1, D)
    Xd = _tc_dup_cast(Xp)

    xep1 = _sc_phase_a(Xd, rows3, cols3, vals3)
    xvp1 = _sc_phase_b(xep1, cols3, rows3, vals3)
    xep2, xeq = _sc_phase_bln(xvp1, Xq, wq0, bq0, rows3, cols3, vals3)
    xvp2 = _sc_phase_b(xep2, cols3, rows3, vals3)
    XoutQ = _sc_final(xvp2, xeq, X0q, wq1, bq1)
    return jnp.take(XoutQ, jnp.asarray(_INVQ), axis=1)[:N]


# speedup vs baseline: 1.1864x; 1.1864x over previous
import jax
import jax.numpy as jnp
from jax import lax
from jax.experimental import pallas as pl
from jax.experimental.pallas import tpu as pltpu
from jax.experimental.pallas import tpu_sc as plsc

N = 10000
D = 128
E = 320000
ALPHA = 0.5
SLOPE = 0.2
EPS = 1e-5

NC = 2
NS = 16
CH = 128
SCCH = 20
NBUF = 5
NP = 10240
EP = NC * NS * 4 * SCCH * CH
NSCP = EP // (SCCH * CH)
SC_PER_CORE = NSCP // NC
ROWS_PER_TILE = NP // NS
ZROWS = 128
BF = jnp.bfloat16


def _phase_common(table, gq, sq, vq, out,
                  gidx_v, sidx_v, val_v, bufs, zbuf, acc, gsem, ssem,
                  c, s, myrow):
    buf0, buf1 = bufs[0], bufs[1]
    buf2 = bufs[2]
    coff = c * NP

    def _zfill(r, _):
        for f in range(D // 32):
            zbuf[r, pl.ds(f * 32, 32)] = jnp.zeros((32,), BF)
        return 0
    lax.fori_loop(0, ZROWS, _zfill, 0)
    for j in range(ROWS_PER_TILE // ZROWS):
        pltpu.sync_copy(zbuf, acc.at[pl.ds(myrow + j * ZROWS, ZROWS)])
    plsc.subcore_barrier()

    def scale(buf, k):
        def sbody(i, _):
            base = pl.multiple_of(i * 16, 16)
            val16 = val_v[k, pl.ds(base, 16)]
            for ee in range(16):
                v16 = jnp.full((16,), val16[ee])
                v = plsc.pack(v16, v16, format=plsc.PackFormat.INTERLEAVED)
                e = base + ee
                for f in range(D // 32):
                    buf[e, pl.ds(f * 32, 32)] = buf[e, pl.ds(f * 32, 32)] * v
            return 0
        lax.fori_loop(0, CH // 16, sbody, 0)

    def outer(j, _):
        sc = c * SC_PER_CORE + j * NS + s
        pltpu.sync_copy(gq.at[sc], gidx_v)
        pltpu.sync_copy(sq.at[sc], sidx_v)
        pltpu.sync_copy(vq.at[sc], val_v)

        def adj(k, _):
            for f in range(CH // 16):
                gidx_v[k, pl.ds(f * 16, 16)] = (
                    gidx_v[k, pl.ds(f * 16, 16)] + coff)
            return 0
        lax.fori_loop(0, SCCH, adj, 0)

        pltpu.make_async_copy(table.at[gidx_v.at[0]], buf0, gsem).start()
        pltpu.make_async_copy(table.at[gidx_v.at[1]], buf1, gsem).start()
        pltpu.make_async_copy(table.at[gidx_v.at[2]], buf2, gsem).start()

        def inner(k4, _):
            for b in range(NBUF):
                k = k4 * NBUF + b
                @pl.when(k >= 2)
                def _():
                    pltpu.make_async_copy(
                        bufs[b], acc.at[sidx_v.at[0]], ssem).wait()

                @pl.when(k < SCCH - 3)
                def _():
                    pltpu.make_async_copy(
                        table.at[gidx_v.at[k + 3]], bufs[(b + 3) % NBUF],
                        gsem).start()
                pltpu.make_async_copy(table.at[gidx_v.at[k]], bufs[b],
                                      gsem).wait()
                scale(bufs[b], k)
                pltpu.make_async_copy(
                    bufs[b], acc.at[sidx_v.at[k]], ssem).start(add=True)
            return 0
        lax.fori_loop(0, SCCH // NBUF, inner, 0)
        pltpu.make_async_copy(buf0, acc.at[sidx_v.at[0]], ssem).wait()
        pltpu.make_async_copy(buf1, acc.at[sidx_v.at[0]], ssem).wait()
        return 0
    lax.fori_loop(0, 4, outer, 0)

    plsc.subcore_barrier()
    pltpu.sync_copy(acc.at[pl.ds(myrow, ROWS_PER_TILE)],
                    out.at[c, pl.ds(myrow, ROWS_PER_TILE)])


def _sc_phase_a_body(table, gq, sq, vq, out,
                     gidx_v, sidx_v, val_v, buf0, buf1, buf2, buf3, buf4,
                     zbuf, acc, gsem, ssem):
    c = lax.axis_index("c")
    s = lax.axis_index("s")
    myrow = s * ROWS_PER_TILE
    _phase_common(table, gq, sq, vq, out,
                  gidx_v, sidx_v, val_v, (buf0, buf1, buf2, buf3, buf4),
                  zbuf, acc, gsem, ssem, c, s, myrow)


def _sc_phase_b_body(p, gq, sq, vq, out, tout,
                     gidx_v, sidx_v, val_v, buf0, buf1, buf2, buf3, buf4,
                     zbuf, acc, gsem, ssem):
    c = lax.axis_index("c")
    s = lax.axis_index("s")
    myrow = s * ROWS_PER_TILE
    for j in range(ROWS_PER_TILE // CH):
        r0 = myrow + j * CH
        pltpu.sync_copy(p.at[0, pl.ds(r0, CH)], buf0)
        pltpu.sync_copy(p.at[1, pl.ds(r0, CH)], buf1)

        def cadd(r, _):
            for f in range(D // 32):
                buf0[r, pl.ds(f * 32, 32)] = (
                    buf0[r, pl.ds(f * 32, 32)] + buf1[r, pl.ds(f * 32, 32)])
            return 0
        lax.fori_loop(0, CH, cadd, 0)
        pltpu.sync_copy(buf0, tout.at[pl.ds(c * NP + r0, CH)])
    _phase_common(tout, gq, sq, vq, out,
                  gidx_v, sidx_v, val_v, (buf0, buf1, buf2, buf3, buf4),
                  zbuf, acc, gsem, ssem, c, s, myrow)


_SCRATCH = [
    pltpu.VMEM((SCCH, CH), jnp.int32),
    pltpu.VMEM((SCCH, CH), jnp.int32),
    pltpu.VMEM((SCCH, CH), jnp.float32),
    pltpu.VMEM((CH, D), BF),
    pltpu.VMEM((CH, D), BF),
    pltpu.VMEM((CH, D), BF),
    pltpu.VMEM((CH, D), BF),
    pltpu.VMEM((CH, D), BF),
    pltpu.VMEM((ZROWS, D), BF),
    pltpu.VMEM_SHARED((NP, D), BF),
    pltpu.SemaphoreType.DMA,
    pltpu.SemaphoreType.DMA,
]


def _mesh():
    return plsc.VectorSubcoreMesh(core_axis_name="c", subcore_axis_name="s",
                                  num_cores=NC, num_subcores=NS)


def _sc_phase_a(table, gq, sq, vq):
    f = pl.kernel(
        _sc_phase_a_body,
        out_type=jax.ShapeDtypeStruct((NC, NP, D), BF),
        mesh=_mesh(),
        scratch_types=list(_SCRATCH),
        compiler_params=pltpu.CompilerParams(use_tc_tiling_on_sc=False,
                                             needs_layout_passes=False),
    )
    return f(table, gq, sq, vq)


def _sc_phase_b(p, gq, sq, vq):
    f = pl.kernel(
        _sc_phase_b_body,
        out_type=(jax.ShapeDtypeStruct((NC, NP, D), BF),
                  jax.ShapeDtypeStruct((NC * NP, D), BF)),
        mesh=_mesh(),
        scratch_types=list(_SCRATCH),
        compiler_params=pltpu.CompilerParams(use_tc_tiling_on_sc=False,
                                             needs_layout_passes=False),
    )
    out, _ = f(p, gq, sq, vq)
    return out


def _leaky_ln(h, w, b):
    h = jnp.where(h >= 0, h, SLOPE * h)
    mu = jnp.mean(h, axis=-1, keepdims=True)
    var = jnp.mean((h - mu) ** 2, axis=-1, keepdims=True)
    return (h - mu) / jnp.sqrt(var + EPS) * w + b


_BM = 1280
_NB = NP // _BM


def _tc_ln1_body(p_ref, x_ref, w_ref, b_ref, oe_ref, od_ref):
    h = p_ref[0].astype(jnp.float32) + p_ref[1].astype(jnp.float32)
    y = _leaky_ln(h, w_ref[...], b_ref[...]) + x_ref[...]
    oe_ref[...] = y
    od_ref[...] = y.astype(BF)


def _tc_ln2_body(p_ref, xe_ref, x0_ref, w_ref, b_ref, o_ref):
    h = p_ref[0].astype(jnp.float32) + p_ref[1].astype(jnp.float32)
    y = _leaky_ln(h, w_ref[...], b_ref[...]) + xe_ref[...]
    o_ref[...] = (1.0 - ALPHA) * y + ALPHA * x0_ref[...]


def _p_spec2():
    return pl.BlockSpec((NC, _BM, D), lambda c, i: (0, i, 0))


def _row_spec2():
    return pl.BlockSpec((_BM, D), lambda c, i: (i, 0))


def _dup_spec2():
    return pl.BlockSpec((_BM, D), lambda c, i: (c * _NB + i, 0))


def _vec_spec2():
    return pl.BlockSpec((1, D), lambda c, i: (0, 0))


def _tc_dup_cast_body(x_ref, o_ref):
    o_ref[...] = x_ref[...].astype(BF)


def _tc_dup_cast(x):
    return pl.pallas_call(
        _tc_dup_cast_body,
        grid=(NC, _NB),
        in_specs=[_row_spec2()],
        out_specs=_dup_spec2(),
        out_shape=jax.ShapeDtypeStruct((NC * NP, D), BF),
    )(x)


def _tc_ln1(p, x, w, b):
    return pl.pallas_call(
        _tc_ln1_body,
        grid=(NC, _NB),
        in_specs=[_p_spec2(), _row_spec2(), _vec_spec2(), _vec_spec2()],
        out_specs=[_row_spec2(), _dup_spec2()],
        out_shape=[jax.ShapeDtypeStruct((NP, D), jnp.float32),
                   jax.ShapeDtypeStruct((NC * NP, D), BF)],
    )(p, x, w.reshape(1, D), b.reshape(1, D))


_BM2 = 1000


def _tc_ln2(p, xe, x0, w, b):
    return pl.pallas_call(
        _tc_ln2_body,
        grid=(1, N // _BM2),
        in_specs=[pl.BlockSpec((NC, _BM2, D), lambda c, i: (0, i, 0)),
                  pl.BlockSpec((_BM2, D), lambda c, i: (i, 0)),
                  pl.BlockSpec((_BM2, D), lambda c, i: (i, 0)),
                  _vec_spec2(), _vec_spec2()],
        out_specs=pl.BlockSpec((_BM2, D), lambda c, i: (i, 0)),
        out_shape=jax.ShapeDtypeStruct((N, D), jnp.float32),
    )(p, xe, x0, w.reshape(1, D), b.reshape(1, D))


def _pad_rows(x):
    return jnp.concatenate([x, jnp.zeros((NP - N, D), jnp.float32)], axis=0)


def _pad_edges_idx(x):
    spread = jnp.arange(EP - E, dtype=jnp.int32) % N
    return jnp.concatenate([x, spread]).reshape(NSCP, SCCH, CH)


def kernel(X, adj_indices, adj_values, X0, ln0_w, ln0_b, ln1_w, ln1_b):
    rows3 = _pad_edges_idx(adj_indices[0])
    cols3 = _pad_edges_idx(adj_indices[1])
    vals3 = jnp.concatenate(
        [adj_values, jnp.zeros((EP - E,), jnp.float32)]).reshape(
            NSCP, SCCH, CH)
    Xp = _pad_rows(X)
    Xd = _tc_dup_cast(Xp)

    xep1 = _sc_phase_a(Xd, rows3, cols3, vals3)
    xvp1 = _sc_phase_b(xep1, cols3, rows3, vals3)
    Xe, Xed = _tc_ln1(xvp1, Xp, ln0_w, ln0_b)
    xep2 = _sc_phase_a(Xed, rows3, cols3, vals3)
    xvp2 = _sc_phase_b(xep2, cols3, rows3, vals3)
    return _tc_ln2(xvp2, Xe, X0, ln1_w, ln1_b)
